# baseline (speedup 1.0000x reference)
import jax
import jax.numpy as jnp
from jax.experimental import pallas as pl

B, S, D = 1, 2048, 768
E = 10
EP = 16
TOP_K = 2
CITY_LEN = 10
CITY_EMB = 32
H = 768
FEAT = D + D // 4 + D // 4 + D // 8 + D // 8
BLK = 256
NBLK = S // BLK

_NEG = -1e30


def _router_kernel(feat_ref, wr_ref, wn_ref, coh_ref, cemb_ref, wce_r_ref,
                   wce_n_ref, br_ref, bn_ref, noise_ref,
                   gate1_ref, gating_ref):
    lane = jax.lax.broadcasted_iota(jnp.int32, (S, EP), 1)
    emask = lane < E

    ce_row = jnp.dot(coh_ref[...], cemb_ref[...],
                     preferred_element_type=jnp.float32)
    feat = feat_ref[...]
    logits = (jnp.dot(feat, wr_ref[...], preferred_element_type=jnp.float32)
              + jnp.dot(ce_row, wce_r_ref[...],
                        preferred_element_type=jnp.float32)
              + br_ref[...])
    nse = (jnp.dot(feat, wn_ref[...], preferred_element_type=jnp.float32)
           + jnp.dot(ce_row, wce_n_ref[...],
                     preferred_element_type=jnp.float32)
           + bn_ref[...])
    std = jnp.maximum(nse, 0.0) + jnp.log1p(jnp.exp(-jnp.abs(nse)))
    noisy = jnp.where(emask, logits + noise_ref[...] * std, _NEG)

    m1 = jnp.max(noisy, axis=1, keepdims=True)
    i1 = jnp.min(jnp.where(noisy == m1, lane, 999), axis=1, keepdims=True)
    noisy2 = jnp.where(lane == i1, _NEG, noisy)
    m2 = jnp.max(noisy2, axis=1, keepdims=True)
    i2 = jnp.min(jnp.where(noisy2 == m2, lane, 999), axis=1, keepdims=True)
    eb = jnp.exp(m2 - m1)
    g1 = 1.0 / (1.0 + eb)
    g2 = eb * g1
    gating_ref[...] = (jnp.where(lane == i1, g1, 0.0)
                       + jnp.where(lane == i2, g2, 0.0))

    lm = jnp.max(jnp.where(emask, logits, _NEG), axis=1, keepdims=True)
    ex = jnp.where(emask, jnp.exp(logits - lm), 0.0)
    gate1_ref[...] = ex / jnp.sum(ex, axis=1, keepdims=True)


def _expert_kernel(gating_ref, x_ref, w1_ref, b1_ref, w2_ref, b2_ref, out_ref):
    e = pl.program_id(0)
    j = pl.program_id(1)
    rows = pl.ds(j * BLK, BLK)

    h = jnp.maximum(
        jnp.dot(x_ref[...], w1_ref[0], preferred_element_type=jnp.float32)
        + b1_ref[0], 0.0)
    y = (jnp.dot(h.astype(jnp.bfloat16), w2_ref[0],
                 preferred_element_type=jnp.float32)
         + b2_ref[0])
    eoh = (jax.lax.broadcasted_iota(jnp.int32, (EP, 1), 0) == e
           ).astype(jnp.float32)
    g = jnp.dot(gating_ref[...], eoh, preferred_element_type=jnp.float32)
    contrib = y * g

    @pl.when(e == 0)
    def _():
        out_ref[rows, :] = contrib

    @pl.when(e != 0)
    def _():
        out_ref[rows, :] += contrib


def kernel(x, city, delta_t_info, delta_dis_info, delta_rg_info,
           delta_entropy_info, city_embeddings, route_W, route_b,
           noise_W, noise_b, W1, b1, W2, b2):
    x2d = x[0]
    feat = jnp.concatenate(
        [x2d, delta_t_info[0], delta_dis_info[0], delta_rg_info[0],
         delta_entropy_info[0]], axis=-1)

    def _padE(a):
        return jnp.pad(a, [(0, 0)] * (a.ndim - 1) + [(0, EP - E)])

    wr = _padE(jnp.concatenate([route_W[:, :D], route_W[:, D + CITY_EMB:]],
                               axis=1).T)
    wn = _padE(jnp.concatenate([noise_W[:, :D], noise_W[:, D + CITY_EMB:]],
                               axis=1).T)
    wce_r = _padE(route_W[:, D:D + CITY_EMB].T)
    wce_n = _padE(noise_W[:, D:D + CITY_EMB].T)
    br = _padE(route_b)[None, :]
    bn = _padE(noise_b)[None, :]
    coh = jax.nn.one_hot(city[0], EP, dtype=jnp.float32)[None, :]
    cemb = jnp.pad(city_embeddings, ((0, EP - CITY_LEN), (0, 0)))
    noise = _padE(jax.random.normal(jax.random.key(42), (S, E),
                                    dtype=jnp.float32))

    gate1, gating = pl.pallas_call(
        _router_kernel,
        out_shape=(jax.ShapeDtypeStruct((S, EP), jnp.float32),
                   jax.ShapeDtypeStruct((S, EP), jnp.float32)),
    )(feat, wr, wn, coh, cemb, wce_r, wce_n, br, bn, noise)

    out = pl.pallas_call(
        _expert_kernel,
        grid=(E, NBLK),
        in_specs=[
            pl.BlockSpec((BLK, EP), lambda e, j: (j, 0)),
            pl.BlockSpec((BLK, D), lambda e, j: (j, 0)),
            pl.BlockSpec((1, D, H), lambda e, j: (e, 0, 0)),
            pl.BlockSpec((1, 1, H), lambda e, j: (e, 0, 0)),
            pl.BlockSpec((1, H, D), lambda e, j: (e, 0, 0)),
            pl.BlockSpec((1, 1, D), lambda e, j: (e, 0, 0)),
        ],
        out_specs=pl.BlockSpec((S, D), lambda e, j: (0, 0)),
        out_shape=jax.ShapeDtypeStruct((S, D), jnp.float32),
    )(gating, x2d.astype(jnp.bfloat16), W1.astype(jnp.bfloat16),
      b1[:, None, :], W2.astype(jnp.bfloat16), b2[:, None, :])

    return (out[None], gate1[:, :E][None])


# speedup vs baseline: 1.2500x; 1.2500x over previous
import jax
import jax.numpy as jnp
from jax.experimental import pallas as pl

B, S, D = 1, 2048, 768
E = 10
EP = 16
TOP_K = 2
CITY_LEN = 10
CITY_EMB = 32
H = 768
FEAT = D + D // 4 + D // 4 + D // 8 + D // 8
BLK = 256
NBLK = S // BLK

_NEG = -1e30


def _router_kernel(feat_ref, wr_ref, wn_ref, coh_ref, cemb_ref, wce_r_ref,
                   wce_n_ref, br_ref, bn_ref, noise_ref,
                   gate1_ref, gating_ref):
    lane = jax.lax.broadcasted_iota(jnp.int32, (S, EP), 1)
    emask = lane < E

    ce_row = jnp.dot(coh_ref[...], cemb_ref[...],
                     preferred_element_type=jnp.float32)
    feat = feat_ref[...]
    logits = (jnp.dot(feat, wr_ref[...], preferred_element_type=jnp.float32)
              + jnp.dot(ce_row, wce_r_ref[...],
                        preferred_element_type=jnp.float32)
              + br_ref[...])
    nse = (jnp.dot(feat, wn_ref[...], preferred_element_type=jnp.float32)
           + jnp.dot(ce_row, wce_n_ref[...],
                     preferred_element_type=jnp.float32)
           + bn_ref[...])
    std = jnp.maximum(nse, 0.0) + jnp.log1p(jnp.exp(-jnp.abs(nse)))
    noisy = jnp.where(emask, logits + noise_ref[...] * std, _NEG)

    m1 = jnp.max(noisy, axis=1, keepdims=True)
    i1 = jnp.min(jnp.where(noisy == m1, lane, 999), axis=1, keepdims=True)
    noisy2 = jnp.where(lane == i1, _NEG, noisy)
    m2 = jnp.max(noisy2, axis=1, keepdims=True)
    i2 = jnp.min(jnp.where(noisy2 == m2, lane, 999), axis=1, keepdims=True)
    eb = jnp.exp(m2 - m1)
    g1 = 1.0 / (1.0 + eb)
    g2 = eb * g1
    gating_ref[...] = (jnp.where(lane == i1, g1, 0.0)
                       + jnp.where(lane == i2, g2, 0.0))

    lm = jnp.max(jnp.where(emask, logits, _NEG), axis=1, keepdims=True)
    ex = jnp.where(emask, jnp.exp(logits - lm), 0.0)
    gate1_ref[...] = ex / jnp.sum(ex, axis=1, keepdims=True)


def _expert_kernel(gating_ref, x_ref, w1_ref, b1_ref, w2_ref, b2_ref, out_ref):
    e = pl.program_id(0)

    h = jnp.maximum(
        jnp.dot(x_ref[...], w1_ref[0], preferred_element_type=jnp.float32)
        + b1_ref[0], 0.0)
    y = (jnp.dot(h.astype(jnp.bfloat16), w2_ref[0],
                 preferred_element_type=jnp.float32)
         + b2_ref[0])
    eoh = (jax.lax.broadcasted_iota(jnp.int32, (EP, 1), 0) == e
           ).astype(jnp.float32)
    g = jnp.dot(gating_ref[...], eoh, preferred_element_type=jnp.float32)
    contrib = y * g

    @pl.when(e == 0)
    def _():
        out_ref[...] = contrib

    @pl.when(e != 0)
    def _():
        out_ref[...] += contrib


def kernel(x, city, delta_t_info, delta_dis_info, delta_rg_info,
           delta_entropy_info, city_embeddings, route_W, route_b,
           noise_W, noise_b, W1, b1, W2, b2):
    x2d = x[0]
    feat = jnp.concatenate(
        [x2d, delta_t_info[0], delta_dis_info[0], delta_rg_info[0],
         delta_entropy_info[0]], axis=-1)

    def _padE(a):
        return jnp.pad(a, [(0, 0)] * (a.ndim - 1) + [(0, EP - E)])

    wr = _padE(jnp.concatenate([route_W[:, :D], route_W[:, D + CITY_EMB:]],
                               axis=1).T)
    wn = _padE(jnp.concatenate([noise_W[:, :D], noise_W[:, D + CITY_EMB:]],
                               axis=1).T)
    wce_r = _padE(route_W[:, D:D + CITY_EMB].T)
    wce_n = _padE(noise_W[:, D:D + CITY_EMB].T)
    br = _padE(route_b)[None, :]
    bn = _padE(noise_b)[None, :]
    coh = jax.nn.one_hot(city[0], EP, dtype=jnp.float32)[None, :]
    cemb = jnp.pad(city_embeddings, ((0, EP - CITY_LEN), (0, 0)))
    noise = _padE(jax.random.normal(jax.random.key(42), (S, E),
                                    dtype=jnp.float32))

    gate1, gating = pl.pallas_call(
        _router_kernel,
        out_shape=(jax.ShapeDtypeStruct((S, EP), jnp.float32),
                   jax.ShapeDtypeStruct((S, EP), jnp.float32)),
    )(feat, wr, wn, coh, cemb, wce_r, wce_n, br, bn, noise)

    out = pl.pallas_call(
        _expert_kernel,
        grid=(E,),
        in_specs=[
            pl.BlockSpec((S, EP), lambda e: (0, 0)),
            pl.BlockSpec((S, D), lambda e: (0, 0)),
            pl.BlockSpec((1, D, H), lambda e: (e, 0, 0)),
            pl.BlockSpec((1, 1, H), lambda e: (e, 0, 0)),
            pl.BlockSpec((1, H, D), lambda e: (e, 0, 0)),
            pl.BlockSpec((1, 1, D), lambda e: (e, 0, 0)),
        ],
        out_specs=pl.BlockSpec((S, D), lambda e: (0, 0)),
        out_shape=jax.ShapeDtypeStruct((S, D), jnp.float32),
    )(gating, x2d.astype(jnp.bfloat16), W1.astype(jnp.bfloat16),
      b1[:, None, :], W2.astype(jnp.bfloat16), b2[:, None, :])

    return (out[None], gate1[:, :E][None])


# speedup vs baseline: 1.8395x; 1.4716x over previous
import jax
import jax.numpy as jnp
from jax.experimental import pallas as pl

B, S, D = 1, 2048, 768
E = 10
TOP_K = 2
CITY_LEN = 10
CITY_EMB = 32
H = 768

_NEG = -1e30

_CDIMS = (((1,), (1,)), ((), ()))


def _router_kernel(x_ref, dt_ref, dis_ref, rg_ref, ent_ref, city_ref,
                   cemb_ref, rw_ref, rb_ref, nw_ref, nb_ref, noise_ref,
                   gate1_ref, gating_ref):
    lane = jax.lax.broadcasted_iota(jnp.int32, (S, E), 1)

    coh = (jax.lax.broadcasted_iota(jnp.int32, (1, CITY_LEN), 1)
           == city_ref[0, 0]).astype(jnp.float32)
    ce = jnp.dot(coh, cemb_ref[...], preferred_element_type=jnp.float32)

    def logits_for(w_ref, b_ref):
        o = jax.lax.dot_general(x_ref[...], w_ref[:, :D], _CDIMS,
                                preferred_element_type=jnp.float32)
        c0 = D
        o += jax.lax.dot_general(ce, w_ref[:, c0:c0 + CITY_EMB], _CDIMS,
                                 preferred_element_type=jnp.float32)
        c0 += CITY_EMB
        o += jax.lax.dot_general(dt_ref[...], w_ref[:, c0:c0 + D // 4],
                                 _CDIMS, preferred_element_type=jnp.float32)
        c0 += D // 4
        o += jax.lax.dot_general(dis_ref[...], w_ref[:, c0:c0 + D // 4],
                                 _CDIMS, preferred_element_type=jnp.float32)
        c0 += D // 4
        o += jax.lax.dot_general(rg_ref[...], w_ref[:, c0:c0 + D // 8],
                                 _CDIMS, preferred_element_type=jnp.float32)
        c0 += D // 8
        o += jax.lax.dot_general(ent_ref[...], w_ref[:, c0:c0 + D // 8],
                                 _CDIMS, preferred_element_type=jnp.float32)
        return o + b_ref[...]

    logits = logits_for(rw_ref, rb_ref)
    nse = logits_for(nw_ref, nb_ref)
    std = jnp.maximum(nse, 0.0) + jnp.log1p(jnp.exp(-jnp.abs(nse)))
    noisy = logits + noise_ref[...] * std

    m1 = jnp.max(noisy, axis=1, keepdims=True)
    i1 = jnp.min(jnp.where(noisy == m1, lane, 999), axis=1, keepdims=True)
    noisy2 = jnp.where(lane == i1, _NEG, noisy)
    m2 = jnp.max(noisy2, axis=1, keepdims=True)
    i2 = jnp.min(jnp.where(noisy2 == m2, lane, 999), axis=1, keepdims=True)
    eb = jnp.exp(m2 - m1)
    g1 = 1.0 / (1.0 + eb)
    g2 = eb * g1
    gating_ref[...] = (jnp.where(lane == i1, g1, 0.0)
                       + jnp.where(lane == i2, g2, 0.0))

    lm = jnp.max(logits, axis=1, keepdims=True)
    ex = jnp.exp(logits - lm)
    gate1_ref[...] = ex / jnp.sum(ex, axis=1, keepdims=True)


def _expert_kernel(gating_ref, x_ref, w1_ref, b1_ref, w2_ref, b2_ref, out_ref):
    e = pl.program_id(0)

    xb = x_ref[...].astype(jnp.bfloat16)
    h = jnp.maximum(
        jnp.dot(xb, w1_ref[0].astype(jnp.bfloat16),
                preferred_element_type=jnp.float32) + b1_ref[0], 0.0)
    y = (jnp.dot(h.astype(jnp.bfloat16), w2_ref[0].astype(jnp.bfloat16),
                 preferred_element_type=jnp.float32)
         + b2_ref[0])
    eoh = (jax.lax.broadcasted_iota(jnp.int32, (E, 1), 0) == e
           ).astype(jnp.float32)
    g = jnp.dot(gating_ref[...], eoh, preferred_element_type=jnp.float32)
    contrib = y * g

    @pl.when(e == 0)
    def _():
        out_ref[...] = contrib

    @pl.when(e != 0)
    def _():
        out_ref[...] += contrib


def kernel(x, city, delta_t_info, delta_dis_info, delta_rg_info,
           delta_entropy_info, city_embeddings, route_W, route_b,
           noise_W, noise_b, W1, b1, W2, b2):
    x2d = x[0]
    noise = jax.random.normal(jax.random.key(42), (S, E), dtype=jnp.float32)

    gate1, gating = pl.pallas_call(
        _router_kernel,
        out_shape=(jax.ShapeDtypeStruct((S, E), jnp.float32),
                   jax.ShapeDtypeStruct((S, E), jnp.float32)),
    )(x2d, delta_t_info[0], delta_dis_info[0], delta_rg_info[0],
      delta_entropy_info[0], city.reshape(1, 1).astype(jnp.int32),
      city_embeddings, route_W, route_b.reshape(1, E), noise_W,
      noise_b.reshape(1, E), noise)

    out = pl.pallas_call(
        _expert_kernel,
        grid=(E,),
        in_specs=[
            pl.BlockSpec((S, E), lambda e: (0, 0)),
            pl.BlockSpec((S, D), lambda e: (0, 0)),
            pl.BlockSpec((1, D, H), lambda e: (e, 0, 0)),
            pl.BlockSpec((1, 1, H), lambda e: (e, 0, 0)),
            pl.BlockSpec((1, H, D), lambda e: (e, 0, 0)),
            pl.BlockSpec((1, 1, D), lambda e: (e, 0, 0)),
        ],
        out_specs=pl.BlockSpec((S, D), lambda e: (0, 0)),
        out_shape=jax.ShapeDtypeStruct((S, D), jnp.float32),
    )(gating, x2d, W1, b1[:, None, :], W2, b2[:, None, :])

    return (out[None], gate1[None])


# speedup vs baseline: 1.9922x; 1.0830x over previous
import jax
import jax.numpy as jnp
import numpy as np
from jax.experimental import pallas as pl
from jax.experimental.pallas import tpu as pltpu

B, S, D = 1, 2048, 768
E = 10
TOP_K = 2
CITY_LEN = 10
CITY_EMB = 32
H = 768

_NEG = -1e30

_CDIMS = (((1,), (1,)), ((), ()))

_NOISE = np.asarray(
    jax.jit(lambda: jax.random.normal(jax.random.key(42), (S, E),
                                      dtype=jnp.float32), backend="cpu")())


def _fused_kernel(x_ref, dt_ref, dis_ref, rg_ref, ent_ref, city_ref,
                  cemb_ref, rw_ref, rb_ref, nw_ref, nb_ref, noise_ref,
                  w1_ref, b1_ref, w2_ref, b2_ref,
                  gate1_ref, out_ref, gating_ref):
    s = pl.program_id(0)

    @pl.when(s == 0)
    def _router():
        lane = jax.lax.broadcasted_iota(jnp.int32, (S, E), 1)

        coh = (jax.lax.broadcasted_iota(jnp.int32, (1, CITY_LEN), 1)
               == city_ref[0, 0]).astype(jnp.float32)
        ce = jnp.dot(coh, cemb_ref[...], preferred_element_type=jnp.float32)

        w = jnp.concatenate([rw_ref[...], nw_ref[...]], axis=0)

        def piece(f, c0, width):
            return jax.lax.dot_general(f, w[:, c0:c0 + width], _CDIMS,
                                       preferred_element_type=jnp.float32)

        o = piece(x_ref[...], 0, D)
        o += piece(ce, D, CITY_EMB)
        o += piece(dt_ref[...], D + CITY_EMB, D // 4)
        o += piece(dis_ref[...], D + CITY_EMB + D // 4, D // 4)
        o += piece(rg_ref[...], D + CITY_EMB + D // 2, D // 8)
        o += piece(ent_ref[...], D + CITY_EMB + D // 2 + D // 8, D // 8)

        logits = o[:, :E] + rb_ref[...]
        nse = o[:, E:] + nb_ref[...]
        std = jnp.maximum(nse, 0.0) + jnp.log1p(jnp.exp(-jnp.abs(nse)))
        noisy = logits + noise_ref[...] * std

        m1 = jnp.max(noisy, axis=1, keepdims=True)
        i1 = jnp.min(jnp.where(noisy == m1, lane, 999), axis=1, keepdims=True)
        noisy2 = jnp.where(lane == i1, _NEG, noisy)
        m2 = jnp.max(noisy2, axis=1, keepdims=True)
        i2 = jnp.min(jnp.where(noisy2 == m2, lane, 999), axis=1, keepdims=True)
        eb = jnp.exp(m2 - m1)
        g1 = 1.0 / (1.0 + eb)
        g2 = eb * g1
        gating_ref[...] = (jnp.where(lane == i1, g1, 0.0)
                           + jnp.where(lane == i2, g2, 0.0))

        lm = jnp.max(logits, axis=1, keepdims=True)
        ex = jnp.exp(logits - lm)
        gate1_ref[...] = ex / jnp.sum(ex, axis=1, keepdims=True)

    @pl.when(s > 0)
    def _expert():
        e = s - 1
        xb = x_ref[...].astype(jnp.bfloat16)
        h = jnp.maximum(
            jnp.dot(xb, w1_ref[0].astype(jnp.bfloat16),
                    preferred_element_type=jnp.float32) + b1_ref[0], 0.0)
        y = (jnp.dot(h.astype(jnp.bfloat16), w2_ref[0].astype(jnp.bfloat16),
                     preferred_element_type=jnp.float32)
             + b2_ref[0])
        eoh = (jax.lax.broadcasted_iota(jnp.int32, (E, 1), 0) == e
               ).astype(jnp.float32)
        g = jnp.dot(gating_ref[...], eoh, preferred_element_type=jnp.float32)
        contrib = y * g

        @pl.when(s == 1)
        def _():
            out_ref[...] = contrib

        @pl.when(s > 1)
        def _():
            out_ref[...] += contrib


def kernel(x, city, delta_t_info, delta_dis_info, delta_rg_info,
           delta_entropy_info, city_embeddings, route_W, route_b,
           noise_W, noise_b, W1, b1, W2, b2):
    x2d = x[0]
    noise = jnp.asarray(_NOISE)

    def wmap(s):
        e = jnp.maximum(s - 1, 0)
        return (e, 0, 0)

    gate1, out = pl.pallas_call(
        _fused_kernel,
        grid=(E + 1,),
        in_specs=[
            pl.BlockSpec((S, D), lambda s: (0, 0)),
            pl.BlockSpec((S, D // 4), lambda s: (0, 0)),
            pl.BlockSpec((S, D // 4), lambda s: (0, 0)),
            pl.BlockSpec((S, D // 8), lambda s: (0, 0)),
            pl.BlockSpec((S, D // 8), lambda s: (0, 0)),
            pl.BlockSpec((1, 1), lambda s: (0, 0)),
            pl.BlockSpec((CITY_LEN, CITY_EMB), lambda s: (0, 0)),
            pl.BlockSpec((E, D + CITY_EMB + 3 * D // 4),
                         lambda s: (0, 0)),
            pl.BlockSpec((1, E), lambda s: (0, 0)),
            pl.BlockSpec((E, D + CITY_EMB + 3 * D // 4),
                         lambda s: (0, 0)),
            pl.BlockSpec((1, E), lambda s: (0, 0)),
            pl.BlockSpec((S, E), lambda s: (0, 0)),
            pl.BlockSpec((1, D, H), wmap),
            pl.BlockSpec((1, 1, H), wmap),
            pl.BlockSpec((1, H, D), wmap),
            pl.BlockSpec((1, 1, D), wmap),
        ],
        out_specs=(pl.BlockSpec((S, E), lambda s: (0, 0)),
                   pl.BlockSpec((S, D), lambda s: (0, 0))),
        out_shape=(jax.ShapeDtypeStruct((S, E), jnp.float32),
                   jax.ShapeDtypeStruct((S, D), jnp.float32)),
        scratch_shapes=[pltpu.VMEM((S, E), jnp.float32)],
    )(x2d, delta_t_info[0], delta_dis_info[0], delta_rg_info[0],
      delta_entropy_info[0], city.reshape(1, 1).astype(jnp.int32),
      city_embeddings, route_W, route_b.reshape(1, E), noise_W,
      noise_b.reshape(1, E), noise, W1, b1[:, None, :], W2, b2[:, None, :])

    return (out[None], gate1[None])
